# baseline (speedup 1.0000x reference)
import functools

import jax
import jax.numpy as jnp
from jax import lax
from jax.experimental import pallas as pl
from jax.experimental.pallas import tpu as pltpu
from jax.experimental.pallas import tpu_sc as plsc

N = 10000
E = 160000
NP = 10240
NC = 2
NS = 16
L = 16
DC = 256
EPS = 1e-4


def _mesh():
    return plsc.VectorSubcoreMesh(core_axis_name="c", subcore_axis_name="s")


_SC_PARAMS = pltpu.CompilerParams(
    needs_layout_passes=False, use_tc_tiling_on_sc=False
)


def _make_deg():
    EPC = E // (NC * NS)
    OPAD = ((EPC + L - 1) // L) * L
    ZB = NP // NS

    @functools.partial(
        pl.kernel,
        mesh=_mesh(),
        out_type=jax.ShapeDtypeStruct((NC, NP), jnp.float32),
        scratch_types=[
            pltpu.VMEM((EPC,), jnp.int32),
            pltpu.VMEM((OPAD,), jnp.float32),
            pltpu.VMEM((ZB,), jnp.float32),
            pltpu.VMEM_SHARED((NP,), jnp.float32),
        ],
        compiler_params=_SC_PARAMS,
    )
    def deg_kernel(dst_hbm, out_hbm, dst_v, ones_v, zeros_v, hist):
        c = lax.axis_index("c")
        s = lax.axis_index("s")
        tb = (c * NS + s) * EPC
        pltpu.sync_copy(dst_hbm.at[pl.ds(tb, EPC)], dst_v)

        def fill_ones(i, carry):
            ones_v[pl.ds(i * L, L)] = jnp.full((L,), 1.0, jnp.float32)
            return carry

        lax.fori_loop(0, OPAD // L, fill_ones, 0)

        def fill_zeros(i, carry):
            zeros_v[pl.ds(i * L, L)] = jnp.zeros((L,), jnp.float32)
            return carry

        lax.fori_loop(0, ZB // L, fill_zeros, 0)
        pltpu.sync_copy(zeros_v, hist.at[pl.ds(s * ZB, ZB)])
        plsc.subcore_barrier()
        pltpu.sync_copy(ones_v.at[pl.ds(0, EPC)], hist.at[dst_v], add=True)
        plsc.subcore_barrier()
        pltpu.sync_copy(hist.at[pl.ds(s * ZB, ZB)], out_hbm.at[c, pl.ds(s * ZB, ZB)])

    return deg_kernel


def _make_scatter():
    PC = 4
    CH = NP // (NC * PC)
    EPT = E // NS
    EB = 2000
    NEB = EPT // EB
    NV = EB // L
    KB = 128
    CAP = EB + KB + L
    RT = CH // NS
    ACC = CH + 8

    @functools.partial(
        pl.kernel,
        mesh=_mesh(),
        out_type=jax.ShapeDtypeStruct((NP, DC), jnp.float32),
        scratch_types=[
            pltpu.VMEM((EB,), jnp.int32),
            pltpu.VMEM((EB,), jnp.int32),
            pltpu.VMEM((CAP,), jnp.int32),
            pltpu.VMEM((CAP,), jnp.int32),
            pltpu.VMEM((KB, DC), jnp.float32),
            pltpu.VMEM((KB, DC), jnp.float32),
            pltpu.VMEM((KB, DC), jnp.float32),
            pltpu.VMEM((KB,), jnp.int32),
            pltpu.VMEM((KB,), jnp.int32),
            pltpu.VMEM((KB,), jnp.int32),
            pltpu.VMEM_SHARED((ACC, DC), jnp.float32),
            pltpu.SemaphoreType.DMA,
            pltpu.SemaphoreType.DMA,
            pltpu.SemaphoreType.DMA,
            pltpu.SemaphoreType.DMA,
            pltpu.SemaphoreType.DMA,
            pltpu.SemaphoreType.DMA,
        ],
        compiler_params=_SC_PARAMS,
    )
    def scat(u_hbm, src_hbm, dst_hbm, out_hbm,
             src_v, dst_v, fsrc, fdst, rows0, rows1, rows2,
             bidx0, bidx1, bidx2, acc,
             gsem0, gsem1, gsem2, asem0, asem1, asem2):
        c = lax.axis_index("c")
        s = lax.axis_index("s")
        iota = lax.iota(jnp.int32, L)
        rows = (rows0, rows1, rows2)
        bidx = (bidx0, bidx1, bidx2)
        gsem = (gsem0, gsem1, gsem2)
        asem = (asem0, asem1, asem2)
        ND = 3

        def g_issue(bi, k):
            pltpu.async_copy(
                u_hbm.at[fsrc.at[pl.ds(bi * KB, KB)]], rows[k], gsem[k])

        def g_wait(bi, k):
            pltpu.make_async_copy(
                u_hbm.at[fsrc.at[pl.ds(bi * KB, KB)]], rows[k],
                gsem[k]).wait()

        def a_issue(k):
            pltpu.async_copy(rows[k], acc.at[bidx[k]], asem[k], add=True)

        def a_wait(k):
            pltpu.make_async_copy(rows[k], acc.at[bidx[k]], asem[k]).wait()

        for p in range(PC):
            base = (c * PC + p) * CH
            with jax.named_scope("sc_init"):
                pltpu.sync_copy(u_hbm.at[pl.ds(base + s * RT, RT)],
                                acc.at[pl.ds(s * RT, RT)])
            with jax.named_scope("sc_bar0"):
                plsc.subcore_barrier()

            def eblock(e, carry):
                with jax.named_scope("sc_eload"):
                    pltpu.sync_copy(
                        src_hbm.at[pl.ds(s * EPT + e * EB, EB)], src_v)
                    pltpu.sync_copy(
                        dst_hbm.at[pl.ds(s * EPT + e * EB, EB)], dst_v)

                def scan_body(i, off):
                    dv = dst_v[pl.ds(i * L, L)]
                    sv = src_v[pl.ds(i * L, L)]
                    m = (dv >= base) & (dv < base + CH)
                    plsc.store_compressed(fsrc.at[pl.ds(off, L)], sv, mask=m)
                    plsc.store_compressed(fdst.at[pl.ds(off, L)], dv - base,
                                          mask=m)
                    return off + jnp.sum(jnp.where(m, 1, 0).astype(jnp.int32))

                with jax.named_scope("sc_scan"):
                    off = lax.fori_loop(0, NV, scan_body, jnp.int32(0))

                    for j in range(KB // L):
                        pos = off + j * L + iota
                        plsc.store_scatter(fdst, [pos],
                                           jnp.full((L,), CH, jnp.int32))
                        plsc.store_scatter(fsrc, [pos],
                                           jnp.zeros((L,), jnp.int32))

                nb = (off + KB - 1) // KB

                with jax.named_scope("sc_batches"):
                    @pl.when(nb > 0)
                    def _():
                        g_issue(0, 0)

                    @pl.when(nb > 1)
                    def _():
                        g_issue(1, 1)

                    def tri(o, carry2):
                        for k in range(ND):
                            bi = o * ND + k
                            k2 = (k + 2) % ND

                            @pl.when(bi < nb)
                            def _():
                                g_wait(bi, k)
                                for j in range(KB // L):
                                    bidx[k][pl.ds(j * L, L)] = (
                                        fdst[pl.ds(bi * KB + j * L, L)])
                                a_issue(k)

                                @pl.when(bi + 2 < nb)
                                def _():
                                    @pl.when(bi >= 1)
                                    def _():
                                        a_wait(k2)
                                    g_issue(bi + 2, k2)
                        return carry2

                    lax.fori_loop(0, (nb + ND - 1) // ND, tri, 0)
                    for k in range(ND):
                        @pl.when(k < nb)
                        def _():
                            a_wait(k)
                return carry

            lax.fori_loop(0, NEB, eblock, 0)
            plsc.subcore_barrier()
            pltpu.sync_copy(acc.at[pl.ds(s * RT, RT)],
                            out_hbm.at[pl.ds(base + s * RT, RT)])
            plsc.subcore_barrier()

    return scat


def _dinv_kernel(ha, hb):
    def body(a_ref, b_ref, o_ref):
        o_ref[...] = lax.rsqrt(a_ref[...] + b_ref[...] + 1.0)

    return pl.pallas_call(
        body, out_shape=jax.ShapeDtypeStruct((NP, 1), jnp.float32)
    )(ha, hb)


def _matmul_dinv(h_parts, W_parts, dinv, dout):
    nin = len(h_parts)
    nout = dout // DC
    BR = 2048

    def body(*refs):
        h_refs = refs[:nin]
        w_refs = refs[nin:2 * nin]
        dv_ref = refs[2 * nin]
        o_refs = refs[2 * nin + 1:]
        acc = jnp.dot(h_refs[0][...], w_refs[0][...],
                      preferred_element_type=jnp.float32)
        for i in range(1, nin):
            acc += jnp.dot(h_refs[i][...], w_refs[i][...],
                           preferred_element_type=jnp.float32)
        acc = acc * dv_ref[...]
        for j in range(nout):
            o_refs[j][...] = acc[:, j * DC:(j + 1) * DC]

    return pl.pallas_call(
        body,
        grid=(NP // BR,),
        in_specs=(
            [pl.BlockSpec((BR, DC), lambda i: (i, 0)) for _ in range(nin)]
            + [pl.BlockSpec((DC, dout), lambda i: (0, 0)) for _ in range(nin)]
            + [pl.BlockSpec((BR, 1), lambda i: (i, 0))]
        ),
        out_specs=[pl.BlockSpec((BR, DC), lambda i: (i, 0))
                   for _ in range(nout)],
        out_shape=[jax.ShapeDtypeStruct((NP, DC), jnp.float32)
                   for _ in range(nout)],
    )(*h_parts, *W_parts, dinv)


def _post(S, dinv, b, g, bt, sig):
    BC = 128

    def body(s_ref, dv_ref, b_ref, g_ref, bt_ref, o_ref):
        t = s_ref[...] * dv_ref[...] + b_ref[...]
        if sig:
            t = jax.nn.sigmoid(t)
        rid = lax.broadcasted_iota(jnp.int32, (NP, 1), 0)
        valid = rid < N
        tm = jnp.where(valid, t, 0.0)
        mu = jnp.sum(tm, axis=0, keepdims=True) * (1.0 / N)
        d = jnp.where(valid, t - mu, 0.0)
        var = jnp.sum(d * d, axis=0, keepdims=True) * (1.0 / N)
        o = (t - mu) * lax.rsqrt(var + EPS) * g_ref[...] + bt_ref[...]
        o_ref[...] = jnp.where(valid, o, 0.0)

    return pl.pallas_call(
        body,
        grid=(DC // BC,),
        in_specs=[
            pl.BlockSpec((NP, BC), lambda i: (0, i)),
            pl.BlockSpec((NP, 1), lambda i: (0, 0)),
            pl.BlockSpec((1, BC), lambda i: (0, i)),
            pl.BlockSpec((1, BC), lambda i: (0, i)),
            pl.BlockSpec((1, BC), lambda i: (0, i)),
        ],
        out_specs=pl.BlockSpec((NP, BC), lambda i: (0, i)),
        out_shape=jax.ShapeDtypeStruct((NP, DC), jnp.float32),
    )(S, dinv, b, g, bt)


def kernel(x, edge_index, W0, b0, g0, bt0, W1, b1, g1, bt1, W2, b2, g2, bt2):
    src = edge_index[0]
    dst = edge_index[1]
    hist = _make_deg()(dst)
    dinv = _dinv_kernel(hist[0].reshape(NP, 1), hist[1].reshape(NP, 1))
    scat = _make_scatter()
    xp = jnp.pad(x, ((0, NP - N), (0, 0)))
    h_parts = tuple(xp[:, i * DC:(i + 1) * DC] for i in range(x.shape[1] // DC))
    params = [(W0, b0, g0, bt0, True), (W1, b1, g1, bt1, True),
              (W2, b2, g2, bt2, False)]
    for W, b, g, bt, sig in params:
        dout = W.shape[1]
        W_parts = tuple(W[i * DC:(i + 1) * DC, :] for i in range(len(h_parts)))
        u_parts = _matmul_dinv(h_parts, W_parts, dinv, dout)
        new_parts = []
        for j in range(dout // DC):
            S = scat(u_parts[j], src, dst)
            cs = slice(j * DC, (j + 1) * DC)
            new_parts.append(
                _post(S, dinv, b[cs].reshape(1, DC), g[cs].reshape(1, DC),
                      bt[cs].reshape(1, DC), sig))
        h_parts = tuple(new_parts)
    return h_parts[0][:N]


# speedup vs baseline: 2.9809x; 2.9809x over previous
import functools

import jax
import jax.numpy as jnp
from jax import lax
from jax.experimental import pallas as pl
from jax.experimental.pallas import tpu as pltpu
from jax.experimental.pallas import tpu_sc as plsc

N = 10000
E = 160000
NP = 10240
NC = 2
NS = 16
L = 16
DC = 256
EPS = 1e-4


def _mesh():
    return plsc.VectorSubcoreMesh(core_axis_name="c", subcore_axis_name="s")


_SC_PARAMS = pltpu.CompilerParams(
    needs_layout_passes=False, use_tc_tiling_on_sc=False
)


def _make_deg():
    EPC = E // (NC * NS)
    OPAD = ((EPC + L - 1) // L) * L
    ZB = NP // NS

    @functools.partial(
        pl.kernel,
        mesh=_mesh(),
        out_type=jax.ShapeDtypeStruct((NC, NP), jnp.float32),
        scratch_types=[
            pltpu.VMEM((EPC,), jnp.int32),
            pltpu.VMEM((OPAD,), jnp.float32),
            pltpu.VMEM((ZB,), jnp.float32),
            pltpu.VMEM_SHARED((NP,), jnp.float32),
        ],
        compiler_params=_SC_PARAMS,
    )
    def deg_kernel(dst_hbm, out_hbm, dst_v, ones_v, zeros_v, hist):
        c = lax.axis_index("c")
        s = lax.axis_index("s")
        tb = (c * NS + s) * EPC
        pltpu.sync_copy(dst_hbm.at[pl.ds(tb, EPC)], dst_v)

        def fill_ones(i, carry):
            ones_v[pl.ds(i * L, L)] = jnp.full((L,), 1.0, jnp.float32)
            return carry

        lax.fori_loop(0, OPAD // L, fill_ones, 0)

        def fill_zeros(i, carry):
            zeros_v[pl.ds(i * L, L)] = jnp.zeros((L,), jnp.float32)
            return carry

        lax.fori_loop(0, ZB // L, fill_zeros, 0)
        pltpu.sync_copy(zeros_v, hist.at[pl.ds(s * ZB, ZB)])
        plsc.subcore_barrier()
        pltpu.sync_copy(ones_v.at[pl.ds(0, EPC)], hist.at[dst_v], add=True)
        plsc.subcore_barrier()
        pltpu.sync_copy(hist.at[pl.ds(s * ZB, ZB)], out_hbm.at[c, pl.ds(s * ZB, ZB)])

    return deg_kernel


def _make_scatter():
    PC = 1
    CH = NP // (NC * PC)
    EPT = E // NS
    EB = 2000
    NEB = EPT // EB
    NV = EB // L
    KB = 128
    CAP = EB + KB + L
    RT = CH // NS
    ACC = CH + 8

    @functools.partial(
        pl.kernel,
        mesh=_mesh(),
        out_type=jax.ShapeDtypeStruct((NP, DC), jnp.float32),
        scratch_types=[
            pltpu.VMEM((EB,), jnp.int32),
            pltpu.VMEM((EB,), jnp.int32),
            pltpu.VMEM((CAP,), jnp.int32),
            pltpu.VMEM((CAP,), jnp.int32),
            pltpu.VMEM((KB, DC), jnp.float32),
            pltpu.VMEM((KB,), jnp.int32),
            pltpu.VMEM_SHARED((ACC, DC), jnp.float32),
            pltpu.SemaphoreType.DMA,
        ],
        compiler_params=_SC_PARAMS,
    )
    def scat(u_hbm, src_hbm, dst_hbm, out_hbm,
             src_v, dst_v, fsrc, fdst, rows0, bidx, acc, gsem0):
        c = lax.axis_index("c")
        s = lax.axis_index("s")
        iota = lax.iota(jnp.int32, L)

        for p in range(PC):
            base = (c * PC + p) * CH
            with jax.named_scope("sc_init"):
                pltpu.sync_copy(u_hbm.at[pl.ds(base + s * RT, RT)],
                                acc.at[pl.ds(s * RT, RT)])
            with jax.named_scope("sc_bar0"):
                plsc.subcore_barrier()

            def eblock(e, carry):
                with jax.named_scope("sc_eload"):
                    pltpu.sync_copy(
                        src_hbm.at[pl.ds(s * EPT + e * EB, EB)], src_v)
                    pltpu.sync_copy(
                        dst_hbm.at[pl.ds(s * EPT + e * EB, EB)], dst_v)

                def scan_body(i, off):
                    dv = dst_v[pl.ds(i * L, L)]
                    sv = src_v[pl.ds(i * L, L)]
                    m = (dv >= base) & (dv < base + CH)
                    plsc.store_compressed(fsrc.at[pl.ds(off, L)], sv, mask=m)
                    plsc.store_compressed(fdst.at[pl.ds(off, L)], dv - base,
                                          mask=m)
                    return off + jnp.sum(jnp.where(m, 1, 0).astype(jnp.int32))

                with jax.named_scope("sc_scan"):
                    off = lax.fori_loop(0, NV, scan_body, jnp.int32(0))

                    for j in range(KB // L):
                        pos = off + j * L + iota
                        plsc.store_scatter(fdst, [pos],
                                           jnp.full((L,), -1, jnp.int32))
                        plsc.store_scatter(fsrc, [pos],
                                           jnp.zeros((L,), jnp.int32))

                nb = (off + KB - 1) // KB

                def batch(bi, carry2):
                    b = bi * KB
                    with jax.named_scope("sc_gather"):
                        pltpu.async_copy(
                            u_hbm.at[fsrc.at[pl.ds(b, KB)]],
                            rows0, gsem0).wait()
                    with jax.named_scope("sc_add"):
                        for j in range(KB // L):
                            bidx[pl.ds(j * L, L)] = fdst[pl.ds(b + j * L, L)]
                        pltpu.sync_copy(
                            rows0, acc.at[plsc.Indices(bidx,
                                                       ignored_value=-1)],
                            add=True)
                    return carry2

                with jax.named_scope("sc_batches"):
                    lax.fori_loop(0, nb, batch, 0)
                return carry

            lax.fori_loop(0, NEB, eblock, 0)
            plsc.subcore_barrier()
            pltpu.sync_copy(acc.at[pl.ds(s * RT, RT)],
                            out_hbm.at[pl.ds(base + s * RT, RT)])
            plsc.subcore_barrier()

    return scat


def _dinv_kernel(ha, hb):
    def body(a_ref, b_ref, o_ref):
        o_ref[...] = lax.rsqrt(a_ref[...] + b_ref[...] + 1.0)

    return pl.pallas_call(
        body, out_shape=jax.ShapeDtypeStruct((NP, 1), jnp.float32)
    )(ha, hb)


def _matmul_dinv(h_parts, W_parts, dinv, dout):
    nin = len(h_parts)
    nout = dout // DC
    BR = 2048

    def body(*refs):
        h_refs = refs[:nin]
        w_refs = refs[nin:2 * nin]
        dv_ref = refs[2 * nin]
        o_refs = refs[2 * nin + 1:]
        acc = jnp.dot(h_refs[0][...], w_refs[0][...],
                      preferred_element_type=jnp.float32)
        for i in range(1, nin):
            acc += jnp.dot(h_refs[i][...], w_refs[i][...],
                           preferred_element_type=jnp.float32)
        acc = acc * dv_ref[...]
        for j in range(nout):
            o_refs[j][...] = acc[:, j * DC:(j + 1) * DC]

    return pl.pallas_call(
        body,
        grid=(NP // BR,),
        in_specs=(
            [pl.BlockSpec((BR, DC), lambda i: (i, 0)) for _ in range(nin)]
            + [pl.BlockSpec((DC, dout), lambda i: (0, 0)) for _ in range(nin)]
            + [pl.BlockSpec((BR, 1), lambda i: (i, 0))]
        ),
        out_specs=[pl.BlockSpec((BR, DC), lambda i: (i, 0))
                   for _ in range(nout)],
        out_shape=[jax.ShapeDtypeStruct((NP, DC), jnp.float32)
                   for _ in range(nout)],
    )(*h_parts, *W_parts, dinv)


def _post(S, dinv, b, g, bt, sig):
    BC = 128

    def body(s_ref, dv_ref, b_ref, g_ref, bt_ref, o_ref):
        t = s_ref[...] * dv_ref[...] + b_ref[...]
        if sig:
            t = jax.nn.sigmoid(t)
        rid = lax.broadcasted_iota(jnp.int32, (NP, 1), 0)
        valid = rid < N
        tm = jnp.where(valid, t, 0.0)
        mu = jnp.sum(tm, axis=0, keepdims=True) * (1.0 / N)
        d = jnp.where(valid, t - mu, 0.0)
        var = jnp.sum(d * d, axis=0, keepdims=True) * (1.0 / N)
        o = (t - mu) * lax.rsqrt(var + EPS) * g_ref[...] + bt_ref[...]
        o_ref[...] = jnp.where(valid, o, 0.0)

    return pl.pallas_call(
        body,
        grid=(DC // BC,),
        in_specs=[
            pl.BlockSpec((NP, BC), lambda i: (0, i)),
            pl.BlockSpec((NP, 1), lambda i: (0, 0)),
            pl.BlockSpec((1, BC), lambda i: (0, i)),
            pl.BlockSpec((1, BC), lambda i: (0, i)),
            pl.BlockSpec((1, BC), lambda i: (0, i)),
        ],
        out_specs=pl.BlockSpec((NP, BC), lambda i: (0, i)),
        out_shape=jax.ShapeDtypeStruct((NP, DC), jnp.float32),
    )(S, dinv, b, g, bt)


def kernel(x, edge_index, W0, b0, g0, bt0, W1, b1, g1, bt1, W2, b2, g2, bt2):
    src = edge_index[0]
    dst = edge_index[1]
    hist = _make_deg()(dst)
    dinv = _dinv_kernel(hist[0].reshape(NP, 1), hist[1].reshape(NP, 1))
    scat = _make_scatter()
    xp = jnp.pad(x, ((0, NP - N), (0, 0)))
    h_parts = tuple(xp[:, i * DC:(i + 1) * DC] for i in range(x.shape[1] // DC))
    params = [(W0, b0, g0, bt0, True), (W1, b1, g1, bt1, True),
              (W2, b2, g2, bt2, False)]
    for W, b, g, bt, sig in params:
        dout = W.shape[1]
        W_parts = tuple(W[i * DC:(i + 1) * DC, :] for i in range(len(h_parts)))
        u_parts = _matmul_dinv(h_parts, W_parts, dinv, dout)
        new_parts = []
        for j in range(dout // DC):
            S = scat(u_parts[j], src, dst)
            cs = slice(j * DC, (j + 1) * DC)
            new_parts.append(
                _post(S, dinv, b[cs].reshape(1, DC), g[cs].reshape(1, DC),
                      bt[cs].reshape(1, DC), sig))
        h_parts = tuple(new_parts)
    return h_parts[0][:N]
